# Initial kernel scaffold; baseline (speedup 1.0000x reference)
import jax
import jax.numpy as jnp
from jax.experimental import pallas as pl


def kernel(hidden_states, Wr, br, w1, w2, w3):
    raise NotImplementedError("write your pallas kernel here")


# speedup vs baseline: 3.2364x; 3.2364x over previous
import functools

import jax
import jax.numpy as jnp
from jax import lax
from jax.experimental import pallas as pl
from jax.experimental.pallas import tpu as pltpu
from jax.experimental.pallas import tpu_sc as plsc

BT = 256
BH = 512
CK = 256


def _router_body(x_ref, wr_ref, br_ref, mi_ref, mp_ref):
    x = x_ref[...]
    wr = wr_ref[...]
    logits = lax.dot_general(x, wr, (((1,), (1,)), ((), ())),
                             preferred_element_type=jnp.float32)
    logits = logits + br_ref[...]
    mx = jnp.max(logits, axis=1, keepdims=True)
    mp_ref[...] = 1.0 / jnp.sum(jnp.exp(logits - mx), axis=1)
    mi_ref[...] = jnp.argmax(logits, axis=1).astype(jnp.int32)


def _meta_body(mi_ref, dest_ref, meta_ref, *, T, E, NB):
    mi = mi_ref[...]
    eio = lax.broadcasted_iota(jnp.int32, (T, E), 1)
    oh = mi[:, None] == eio
    ohf = oh.astype(jnp.float32)
    ohb = oh.astype(jnp.bfloat16)
    counts = jnp.sum(ohf, axis=0, keepdims=True)
    nb = jnp.floor((counts + (BT - 1)) * (1.0 / BT))
    t8 = (lax.broadcasted_iota(jnp.int32, (E, E), 0)
          <= lax.broadcasted_iota(jnp.int32, (E, E), 1)).astype(jnp.float32)
    cnb = lax.dot_general(nb, t8, (((1,), (0,)), ((), ())),
                          preferred_element_type=jnp.float32)
    poff = (cnb - nb) * BT
    for k in range(T // CK):
        rio = lax.broadcasted_iota(jnp.int32, (CK, T), 0) + k * CK
        cio = lax.broadcasted_iota(jnp.int32, (CK, T), 1)
        ltri = (rio > cio).astype(jnp.bfloat16)
        cum = lax.dot_general(ltri, ohb, (((1,), (0,)), ((), ())),
                              preferred_element_type=jnp.float32)
        destk = jnp.sum(ohf[k * CK:(k + 1) * CK] * (poff + cum), axis=1)
        dest_ref[pl.ds(k * CK, CK)] = destk.astype(jnp.int32)
    jio = lax.broadcasted_iota(jnp.int32, (128, E), 0).astype(jnp.float32)
    be = jnp.sum((jnp.broadcast_to(cnb, (128, E)) <= jio).astype(jnp.float32),
                 axis=1)
    be = jnp.minimum(be, float(E - 1))
    nvalid = jnp.sum(nb)
    lio = lax.broadcasted_iota(jnp.int32, (128,), 0)
    meta_ref[...] = jnp.where(lio == NB, nvalid, be).astype(jnp.int32)


def _scatter_body(x_hbm, dest_hbm, mp_hbm, xs_hbm, mp16_hbm,
                  idx_v, rows_v, mpv, mp16_v, sem, *, rows_per_chunk, chunks):
    nc = 2
    wid = lax.axis_index("s") * nc + lax.axis_index("c")
    base = wid * rows_per_chunk * chunks
    for c in range(chunks):
        lo = base + c * rows_per_chunk
        pltpu.sync_copy(dest_hbm.at[pl.ds(lo, rows_per_chunk)], idx_v.at[c])
        pltpu.sync_copy(x_hbm.at[pl.ds(lo, rows_per_chunk)], rows_v)
        pltpu.async_copy(rows_v, xs_hbm.at[idx_v.at[c]], sem).wait()
        pltpu.sync_copy(mp_hbm.at[pl.ds(lo, rows_per_chunk)], mpv)
        for g in range(rows_per_chunk // 16):
            vals = mpv[pl.ds(g * 16, 16)]
            rid = lax.iota(jnp.int32, 16) + g * 16
            plsc.store_scatter(mp16_v, [rid, jnp.zeros((16,), jnp.int32)],
                               vals)
        pltpu.async_copy(mp16_v, mp16_hbm.at[idx_v.at[c]], sem).wait()


def _mlp_body(meta_ref, xs_ref, w1_ref, w2_ref, w3_ref, mp_ref, y_ref,
              acc_ref, *, NB, NH):
    j = pl.program_id(0)
    h = pl.program_id(1)
    nvalid = meta_ref[NB]

    @pl.when(j < nvalid)
    def _():
        xb = xs_ref[...]
        w1b = w1_ref[0]
        w2b = w2_ref[0]
        g = lax.dot_general(xb, w1b, (((1,), (1,)), ((), ())),
                            preferred_element_type=jnp.float32)
        u = lax.dot_general(xb, w2b, (((1,), (1,)), ((), ())),
                            preferred_element_type=jnp.float32)
        z = g * lax.logistic(g) * u
        w3b = w3_ref[0]
        part = lax.dot_general(z, w3b, (((1,), (1,)), ((), ())),
                               preferred_element_type=jnp.float32)

        @pl.when(h == 0)
        def _():
            acc_ref[...] = part

        @pl.when(h > 0)
        def _():
            acc_ref[...] += part

        @pl.when(h == NH - 1)
        def _():
            mpc = mp_ref[:, 0:1]
            y_ref[...] = acc_ref[...].astype(jnp.bfloat16).astype(
                jnp.float32) * mpc


def _gather_body(y_hbm, dest_hbm, out_hbm, idx_v, rows_v, sem,
                 *, rows_per_chunk, chunks):
    nc = 2
    wid = lax.axis_index("s") * nc + lax.axis_index("c")
    base = wid * rows_per_chunk * chunks
    for c in range(chunks):
        lo = base + c * rows_per_chunk
        pltpu.sync_copy(dest_hbm.at[pl.ds(lo, rows_per_chunk)], idx_v.at[c])
        pltpu.async_copy(y_hbm.at[idx_v.at[c]], rows_v, sem).wait()
        pltpu.sync_copy(rows_v, out_hbm.at[pl.ds(lo, rows_per_chunk)])


def kernel(hidden_states, Wr, br, w1, w2, w3):
    s, b, d = hidden_states.shape
    e, hid, _ = w1.shape
    T = s * b
    NH = hid // BH
    NB = T // BT + e - 1
    P = NB * BT

    x2d = hidden_states.reshape(T, d)

    mi, mp = pl.pallas_call(
        _router_body,
        out_shape=(jax.ShapeDtypeStruct((T,), jnp.int32),
                   jax.ShapeDtypeStruct((T,), jnp.float32)),
    )(x2d, Wr, br.reshape(1, e))

    dest, meta128 = pl.pallas_call(
        functools.partial(_meta_body, T=T, E=e, NB=NB),
        out_shape=(jax.ShapeDtypeStruct((T,), jnp.int32),
                   jax.ShapeDtypeStruct((128,), jnp.int32)),
    )(mi)
    meta_vec = meta128[:NB + 1]

    mesh = plsc.VectorSubcoreMesh(core_axis_name="c", subcore_axis_name="s")
    nw = 32
    rpc = 32
    chunks = T // (nw * rpc)

    x_sorted, mp16 = pl.kernel(
        functools.partial(_scatter_body, rows_per_chunk=rpc, chunks=chunks),
        out_type=(jax.ShapeDtypeStruct((P, d), jnp.float32),
                  jax.ShapeDtypeStruct((P, 128), jnp.float32)),
        mesh=mesh,
        scratch_types=(
            pltpu.VMEM((chunks, rpc), jnp.int32),
            pltpu.VMEM((rpc, d), jnp.float32),
            pltpu.VMEM((rpc,), jnp.float32),
            pltpu.VMEM((rpc, 128), jnp.float32),
            pltpu.SemaphoreType.DMA,
        ),
        compiler_params=pltpu.CompilerParams(needs_layout_passes=False),
    )(x2d, dest, mp)

    y_sorted = pl.pallas_call(
        functools.partial(_mlp_body, NB=NB, NH=NH),
        grid_spec=pltpu.PrefetchScalarGridSpec(
            num_scalar_prefetch=1,
            grid=(NB, NH),
            in_specs=[
                pl.BlockSpec((BT, d), lambda j, h, m: (j, 0)),
                pl.BlockSpec((1, BH, d), lambda j, h, m: (m[j], h, 0)),
                pl.BlockSpec((1, BH, d), lambda j, h, m: (m[j], h, 0)),
                pl.BlockSpec((1, d, BH), lambda j, h, m: (m[j], 0, h)),
                pl.BlockSpec((BT, 128), lambda j, h, m: (j, 0)),
            ],
            out_specs=pl.BlockSpec((BT, d), lambda j, h, m: (j, 0)),
            scratch_shapes=[pltpu.VMEM((BT, d), jnp.float32)],
        ),
        out_shape=jax.ShapeDtypeStruct((P, d), jnp.float32),
        compiler_params=pltpu.CompilerParams(
            dimension_semantics=("arbitrary", "arbitrary")),
    )(meta_vec, x_sorted, w1, w2, w3, mp16)

    out2d = pl.kernel(
        functools.partial(_gather_body, rows_per_chunk=rpc, chunks=chunks),
        out_type=jax.ShapeDtypeStruct((T, d), jnp.float32),
        mesh=mesh,
        scratch_types=(
            pltpu.VMEM((chunks, rpc), jnp.int32),
            pltpu.VMEM((rpc, d), jnp.float32),
            pltpu.SemaphoreType.DMA,
        ),
    )(y_sorted, dest)

    return out2d.reshape(s, b, d)


# speedup vs baseline: 4.6137x; 1.4255x over previous
import functools

import jax
import jax.numpy as jnp
from jax import lax
from jax.experimental import pallas as pl
from jax.experimental.pallas import tpu as pltpu
from jax.experimental.pallas import tpu_sc as plsc

BT = 512
BH = 512
CK = 256


def _router_body(x_ref, wr_ref, br_ref, mi_ref, mp_ref):
    x = x_ref[...]
    wr = wr_ref[...]
    logits = lax.dot_general(x, wr, (((1,), (1,)), ((), ())),
                             preferred_element_type=jnp.float32)
    logits = logits + br_ref[...]
    mx = jnp.max(logits, axis=1, keepdims=True)
    mp_ref[...] = 1.0 / jnp.sum(jnp.exp(logits - mx), axis=1)
    mi_ref[...] = jnp.argmax(logits, axis=1).astype(jnp.int32)


def _meta_body(mi_ref, dest_ref, meta_ref, *, T, E, NB):
    mi = mi_ref[...]
    eio = lax.broadcasted_iota(jnp.int32, (T, E), 1)
    oh = mi[:, None] == eio
    ohf = oh.astype(jnp.float32)
    ohb = oh.astype(jnp.bfloat16)
    counts = jnp.sum(ohf, axis=0, keepdims=True)
    nb = jnp.floor((counts + (BT - 1)) * (1.0 / BT))
    t8 = (lax.broadcasted_iota(jnp.int32, (E, E), 0)
          <= lax.broadcasted_iota(jnp.int32, (E, E), 1)).astype(jnp.float32)
    cnb = lax.dot_general(nb, t8, (((1,), (0,)), ((), ())),
                          preferred_element_type=jnp.float32)
    poff = (cnb - nb) * BT
    for k in range(T // CK):
        rio = lax.broadcasted_iota(jnp.int32, (CK, T), 0) + k * CK
        cio = lax.broadcasted_iota(jnp.int32, (CK, T), 1)
        ltri = (rio > cio).astype(jnp.bfloat16)
        cum = lax.dot_general(ltri, ohb, (((1,), (0,)), ((), ())),
                              preferred_element_type=jnp.float32)
        destk = jnp.sum(ohf[k * CK:(k + 1) * CK] * (poff + cum), axis=1)
        dest_ref[pl.ds(k * CK, CK)] = destk.astype(jnp.int32)
    jio = lax.broadcasted_iota(jnp.int32, (128, E), 0).astype(jnp.float32)
    be = jnp.sum((jnp.broadcast_to(cnb, (128, E)) <= jio).astype(jnp.float32),
                 axis=1)
    be = jnp.minimum(be, float(E - 1))
    nvalid = jnp.sum(nb)
    lio = lax.broadcasted_iota(jnp.int32, (128,), 0)
    meta_ref[...] = jnp.where(lio == NB, nvalid, be).astype(jnp.int32)


def _scatter_body(x_hbm, dest_hbm, mp_hbm, xs_hbm, mp16_hbm,
                  idx_v, rows_v, mpv, mp16_v, sem, *, rows_per_chunk, chunks):
    nc = 2
    wid = lax.axis_index("s") * nc + lax.axis_index("c")
    base = wid * rows_per_chunk * chunks
    for c in range(chunks):
        lo = base + c * rows_per_chunk
        pltpu.sync_copy(dest_hbm.at[pl.ds(lo, rows_per_chunk)], idx_v.at[c])
        pltpu.sync_copy(x_hbm.at[pl.ds(lo, rows_per_chunk)], rows_v)
        pltpu.async_copy(rows_v, xs_hbm.at[idx_v.at[c]], sem).wait()
        pltpu.sync_copy(mp_hbm.at[pl.ds(lo, rows_per_chunk)], mpv)
        for g in range(rows_per_chunk // 16):
            vals = mpv[pl.ds(g * 16, 16)]
            rid = lax.iota(jnp.int32, 16) + g * 16
            plsc.store_scatter(mp16_v, [rid, jnp.zeros((16,), jnp.int32)],
                               vals)
        pltpu.async_copy(mp16_v, mp16_hbm.at[idx_v.at[c]], sem).wait()


def _mlp_body(meta_ref, xs_ref, w1_ref, w2_ref, w3_ref, mp_ref, y_ref,
              acc_ref, *, NB, NH):
    j = pl.program_id(0)
    h = pl.program_id(1)
    nvalid = meta_ref[NB]

    @pl.when(j < nvalid)
    def _():
        xb = xs_ref[...]
        w1b = w1_ref[0]
        w2b = w2_ref[0]
        g = lax.dot_general(xb, w1b, (((1,), (1,)), ((), ())),
                            preferred_element_type=jnp.float32)
        u = lax.dot_general(xb, w2b, (((1,), (1,)), ((), ())),
                            preferred_element_type=jnp.float32)
        z = g * lax.logistic(g) * u
        w3b = w3_ref[0]
        part = lax.dot_general(z, w3b, (((1,), (1,)), ((), ())),
                               preferred_element_type=jnp.float32)

        @pl.when(h == 0)
        def _():
            acc_ref[...] = part

        @pl.when(h > 0)
        def _():
            acc_ref[...] += part

        @pl.when(h == NH - 1)
        def _():
            mpc = mp_ref[:, 0:1]
            y_ref[...] = acc_ref[...].astype(jnp.bfloat16).astype(
                jnp.float32) * mpc


def _gather_body(y_hbm, dest_hbm, out_hbm, idx_v, rows_v, sem,
                 *, rows_per_chunk, chunks):
    nc = 2
    wid = lax.axis_index("s") * nc + lax.axis_index("c")
    base = wid * rows_per_chunk * chunks
    for c in range(chunks):
        lo = base + c * rows_per_chunk
        pltpu.sync_copy(dest_hbm.at[pl.ds(lo, rows_per_chunk)], idx_v.at[c])
        pltpu.async_copy(y_hbm.at[idx_v.at[c]], rows_v, sem).wait()
        pltpu.sync_copy(rows_v, out_hbm.at[pl.ds(lo, rows_per_chunk)])


def kernel(hidden_states, Wr, br, w1, w2, w3):
    s, b, d = hidden_states.shape
    e, hid, _ = w1.shape
    T = s * b
    NH = hid // BH
    NB = T // BT + e - 1
    P = NB * BT

    x2d = hidden_states.reshape(T, d)

    mi, mp = pl.pallas_call(
        _router_body,
        out_shape=(jax.ShapeDtypeStruct((T,), jnp.int32),
                   jax.ShapeDtypeStruct((T,), jnp.float32)),
    )(x2d, Wr, br.reshape(1, e))

    dest, meta128 = pl.pallas_call(
        functools.partial(_meta_body, T=T, E=e, NB=NB),
        out_shape=(jax.ShapeDtypeStruct((T,), jnp.int32),
                   jax.ShapeDtypeStruct((128,), jnp.int32)),
    )(mi)
    meta_vec = meta128[:NB + 1]

    mesh = plsc.VectorSubcoreMesh(core_axis_name="c", subcore_axis_name="s")
    nw = 32
    rpc = 32
    chunks = T // (nw * rpc)

    x_sorted, mp16 = pl.kernel(
        functools.partial(_scatter_body, rows_per_chunk=rpc, chunks=chunks),
        out_type=(jax.ShapeDtypeStruct((P, d), jnp.float32),
                  jax.ShapeDtypeStruct((P, 128), jnp.float32)),
        mesh=mesh,
        scratch_types=(
            pltpu.VMEM((chunks, rpc), jnp.int32),
            pltpu.VMEM((rpc, d), jnp.float32),
            pltpu.VMEM((rpc,), jnp.float32),
            pltpu.VMEM((rpc, 128), jnp.float32),
            pltpu.SemaphoreType.DMA,
        ),
        compiler_params=pltpu.CompilerParams(needs_layout_passes=False),
    )(x2d, dest, mp)

    def _row_idx(j, h, m):
        return (jnp.minimum(j, m[NB] - 1), 0)

    def _w12_idx(j, h, m):
        valid = j < m[NB]
        jj = jnp.where(valid, j, m[NB] - 1)
        hh = jnp.where(valid, h, NH - 1)
        return (m[jj], hh, 0)

    def _w3_idx(j, h, m):
        valid = j < m[NB]
        jj = jnp.where(valid, j, m[NB] - 1)
        hh = jnp.where(valid, h, NH - 1)
        return (m[jj], 0, hh)

    y_sorted = pl.pallas_call(
        functools.partial(_mlp_body, NB=NB, NH=NH),
        grid_spec=pltpu.PrefetchScalarGridSpec(
            num_scalar_prefetch=1,
            grid=(NB, NH),
            in_specs=[
                pl.BlockSpec((BT, d), _row_idx),
                pl.BlockSpec((1, BH, d), _w12_idx),
                pl.BlockSpec((1, BH, d), _w12_idx),
                pl.BlockSpec((1, d, BH), _w3_idx),
                pl.BlockSpec((BT, 128), _row_idx),
            ],
            out_specs=pl.BlockSpec((BT, d), _row_idx),
            scratch_shapes=[pltpu.VMEM((BT, d), jnp.float32)],
        ),
        out_shape=jax.ShapeDtypeStruct((P, d), jnp.float32),
        compiler_params=pltpu.CompilerParams(
            dimension_semantics=("arbitrary", "arbitrary")),
    )(meta_vec, x_sorted, w1, w2, w3, mp16)

    out2d = pl.kernel(
        functools.partial(_gather_body, rows_per_chunk=rpc, chunks=chunks),
        out_type=jax.ShapeDtypeStruct((T, d), jnp.float32),
        mesh=mesh,
        scratch_types=(
            pltpu.VMEM((chunks, rpc), jnp.int32),
            pltpu.VMEM((rpc, d), jnp.float32),
            pltpu.SemaphoreType.DMA,
        ),
    )(y_sorted, dest)

    return out2d.reshape(s, b, d)


# speedup vs baseline: 4.9528x; 1.0735x over previous
import functools

import jax
import jax.numpy as jnp
from jax import lax
from jax.experimental import pallas as pl
from jax.experimental.pallas import tpu as pltpu
from jax.experimental.pallas import tpu_sc as plsc

BT = 512
BH = 512
CK = 256


def _router_meta_body(x_ref, wr_ref, br_ref, mp_ref, dest_ref, meta_ref,
                      *, T, E, NB):
    x = x_ref[...].reshape(T, -1)
    wr = wr_ref[...]
    logits = lax.dot_general(x, wr, (((1,), (1,)), ((), ())),
                             preferred_element_type=jnp.float32)
    logits = logits + br_ref[...]
    mx = jnp.max(logits, axis=1, keepdims=True)
    mp_ref[...] = 1.0 / jnp.sum(jnp.exp(logits - mx), axis=1)
    mi = jnp.argmax(logits, axis=1).astype(jnp.int32)

    eio = lax.broadcasted_iota(jnp.int32, (T, E), 1)
    oh = mi[:, None] == eio
    ohf = oh.astype(jnp.float32)
    ohb = oh.astype(jnp.bfloat16)
    counts = jnp.sum(ohf, axis=0, keepdims=True)
    nb = jnp.floor((counts + (BT - 1)) * (1.0 / BT))
    t8 = (lax.broadcasted_iota(jnp.int32, (E, E), 0)
          <= lax.broadcasted_iota(jnp.int32, (E, E), 1)).astype(jnp.float32)
    cnb = lax.dot_general(nb, t8, (((1,), (0,)), ((), ())),
                          preferred_element_type=jnp.float32)
    poff = (cnb - nb) * BT
    for k in range(T // CK):
        rio = lax.broadcasted_iota(jnp.int32, (CK, T), 0) + k * CK
        cio = lax.broadcasted_iota(jnp.int32, (CK, T), 1)
        ltri = (rio > cio).astype(jnp.bfloat16)
        cum = lax.dot_general(ltri, ohb, (((1,), (0,)), ((), ())),
                              preferred_element_type=jnp.float32)
        destk = jnp.sum(ohf[k * CK:(k + 1) * CK] * (poff + cum), axis=1)
        dest_ref[pl.ds(k * CK, CK)] = destk.astype(jnp.int32)
    jio = lax.broadcasted_iota(jnp.int32, (128, E), 0).astype(jnp.float32)
    be = jnp.sum((jnp.broadcast_to(cnb, (128, E)) <= jio).astype(jnp.float32),
                 axis=1)
    be = jnp.minimum(be, float(E - 1))
    nvalid = jnp.sum(nb)
    bej = jnp.clip(be[:, None], 0.0, float(E - 1))
    sel = (lax.broadcasted_iota(jnp.int32, (128, E), 1).astype(jnp.float32)
           == bej).astype(jnp.float32)
    cnt_j = jnp.sum(sel * counts, axis=1)
    off_j = jnp.sum(sel * poff, axis=1)
    jiof = lax.broadcasted_iota(jnp.int32, (128,), 0).astype(jnp.float32)
    vrows = jnp.clip(cnt_j - (jiof * BT - off_j), 0.0, float(BT))
    lio = lax.broadcasted_iota(jnp.int32, (128,), 0)
    out = jnp.where(lio == NB, nvalid, be)
    out = jnp.where(lio >= 64, vrows, out)
    meta_ref[...] = out.astype(jnp.int32)


def _scatter_body(x_hbm, dest_hbm, mp_hbm, xs_hbm, mp16_hbm,
                  idx_v, rows_v, mpv, mp16_v, sem, *, rows_per_chunk, chunks):
    nc = 2
    wid = lax.axis_index("s") * nc + lax.axis_index("c")
    base = wid * rows_per_chunk * chunks
    for c in range(chunks):
        lo = base + c * rows_per_chunk
        pltpu.sync_copy(dest_hbm.at[pl.ds(lo, rows_per_chunk)], idx_v.at[c])
        pltpu.sync_copy(x_hbm.at[pl.ds(lo, rows_per_chunk), 0], rows_v)
        pltpu.async_copy(rows_v, xs_hbm.at[idx_v.at[c]], sem).wait()
        pltpu.sync_copy(mp_hbm.at[pl.ds(lo, rows_per_chunk)], mpv)
        for g in range(rows_per_chunk // 16):
            vals = mpv[pl.ds(g * 16, 16)]
            rid = lax.iota(jnp.int32, 16) + g * 16
            plsc.store_scatter(mp16_v, [rid, jnp.zeros((16,), jnp.int32)],
                               vals)
        pltpu.async_copy(mp16_v, mp16_hbm.at[idx_v.at[c]], sem).wait()


def _mlp_body(meta_ref, xs_ref, w1_ref, w2_ref, w3_ref, mp_ref, y_ref,
              acc_ref, *, NB, NH):
    j = pl.program_id(0)
    h = pl.program_id(1)
    nvalid = meta_ref[NB]

    @pl.when(j < nvalid)
    def _():
        xb = xs_ref[...]
        w1b = w1_ref[0]
        w2b = w2_ref[0]
        g = lax.dot_general(xb, w1b, (((1,), (1,)), ((), ())),
                            preferred_element_type=jnp.float32)
        u = lax.dot_general(xb, w2b, (((1,), (1,)), ((), ())),
                            preferred_element_type=jnp.float32)
        z = g * lax.logistic(g) * u
        w3b = w3_ref[0]
        part = lax.dot_general(z, w3b, (((1,), (1,)), ((), ())),
                               preferred_element_type=jnp.float32)

        @pl.when(h == 0)
        def _():
            acc_ref[...] = part

        @pl.when(h > 0)
        def _():
            acc_ref[...] += part

        @pl.when(h == NH - 1)
        def _():
            mpc = mp_ref[:, 0:1]
            y_ref[...] = acc_ref[...].astype(jnp.bfloat16).astype(
                jnp.float32) * mpc


def _gather_body(y_hbm, dest_hbm, out_hbm, idx_v, rows_v, sem,
                 *, rows_per_chunk, chunks):
    nc = 2
    wid = lax.axis_index("s") * nc + lax.axis_index("c")
    base = wid * rows_per_chunk * chunks
    for c in range(chunks):
        lo = base + c * rows_per_chunk
        pltpu.sync_copy(dest_hbm.at[pl.ds(lo, rows_per_chunk)], idx_v.at[c])
        pltpu.async_copy(y_hbm.at[idx_v.at[c]], rows_v, sem).wait()
        pltpu.sync_copy(rows_v, out_hbm.at[pl.ds(lo, rows_per_chunk), 0])


def kernel(hidden_states, Wr, br, w1, w2, w3):
    s, b, d = hidden_states.shape
    e, hid, _ = w1.shape
    T = s * b
    NH = hid // BH
    NB = T // BT + e - 1
    P = NB * BT

    mp, dest, meta128 = pl.pallas_call(
        functools.partial(_router_meta_body, T=T, E=e, NB=NB),
        out_shape=(jax.ShapeDtypeStruct((T,), jnp.float32),
                   jax.ShapeDtypeStruct((T,), jnp.int32),
                   jax.ShapeDtypeStruct((128,), jnp.int32)),
    )(hidden_states, Wr, br.reshape(1, e))
    meta_vec = meta128

    mesh = plsc.VectorSubcoreMesh(core_axis_name="c", subcore_axis_name="s")
    nw = 32
    rpc = 32
    chunks = T // (nw * rpc)

    x_sorted, mp16 = pl.kernel(
        functools.partial(_scatter_body, rows_per_chunk=rpc, chunks=chunks),
        out_type=(jax.ShapeDtypeStruct((P, d), jnp.float32),
                  jax.ShapeDtypeStruct((P, 128), jnp.float32)),
        mesh=mesh,
        scratch_types=(
            pltpu.VMEM((chunks, rpc), jnp.int32),
            pltpu.VMEM((rpc, d), jnp.float32),
            pltpu.VMEM((rpc,), jnp.float32),
            pltpu.VMEM((rpc, 128), jnp.float32),
            pltpu.SemaphoreType.DMA,
        ),
        compiler_params=pltpu.CompilerParams(needs_layout_passes=False),
    )(hidden_states, dest, mp)

    def _row_idx(j, h, m):
        return (jnp.minimum(j, m[NB] - 1), 0)

    def _w12_idx(j, h, m):
        valid = j < m[NB]
        jj = jnp.where(valid, j, m[NB] - 1)
        hh = jnp.where(valid, h, NH - 1)
        return (m[jj], hh, 0)

    def _w3_idx(j, h, m):
        valid = j < m[NB]
        jj = jnp.where(valid, j, m[NB] - 1)
        hh = jnp.where(valid, h, NH - 1)
        return (m[jj], 0, hh)

    y_sorted = pl.pallas_call(
        functools.partial(_mlp_body, NB=NB, NH=NH),
        grid_spec=pltpu.PrefetchScalarGridSpec(
            num_scalar_prefetch=1,
            grid=(NB, NH),
            in_specs=[
                pl.BlockSpec((BT, d), _row_idx),
                pl.BlockSpec((1, BH, d), _w12_idx),
                pl.BlockSpec((1, BH, d), _w12_idx),
                pl.BlockSpec((1, d, BH), _w3_idx),
                pl.BlockSpec((BT, 128), _row_idx),
            ],
            out_specs=pl.BlockSpec((BT, d), _row_idx),
            scratch_shapes=[pltpu.VMEM((BT, d), jnp.float32)],
        ),
        out_shape=jax.ShapeDtypeStruct((P, d), jnp.float32),
        compiler_params=pltpu.CompilerParams(
            dimension_semantics=("arbitrary", "arbitrary")),
    )(meta_vec, x_sorted, w1, w2, w3, mp16)

    out = pl.kernel(
        functools.partial(_gather_body, rows_per_chunk=rpc, chunks=chunks),
        out_type=jax.ShapeDtypeStruct((s, b, d), jnp.float32),
        mesh=mesh,
        scratch_types=(
            pltpu.VMEM((chunks, rpc), jnp.int32),
            pltpu.VMEM((rpc, d), jnp.float32),
            pltpu.SemaphoreType.DMA,
        ),
    )(y_sorted, dest)

    return out
